# Initial kernel scaffold; baseline (speedup 1.0000x reference)
import jax
import jax.numpy as jnp
from jax.experimental import pallas as pl


def kernel(H, X_node):
    raise NotImplementedError("write your pallas kernel here")


# speedup vs baseline: 2.7186x; 2.7186x over previous
import jax
import jax.numpy as jnp
from jax import lax
from jax.experimental import pallas as pl
from jax.experimental.pallas import tpu as pltpu
from jax.experimental.pallas import tpu_sc as plsc

E = 320000
V = 10000
D = 128
NC = 2
NS = 16
L = 16

CH = 512
NCH = E // CH
IR = CH // 128
SLOTS = (NCH + NS - 1) // NS

HV = V // NC
ACC_R = 5120
ZR = 64
PT = ACC_R // NS
OT = HV // NS

_mesh = plsc.VectorSubcoreMesh(
    core_axis_name="c", subcore_axis_name="s", num_cores=NC, num_subcores=NS
)


def _body(h_hbm, x2_hbm, out_hbm, acc, hbuf, ibuf, zbuf):
  c = lax.axis_index("c")
  s = lax.axis_index("s")
  lo = c * HV
  hi = lo + HV

  def _zero_zbuf(i, _):
    for k in range(D // L):
      zbuf[i, pl.ds(k * L, L)] = jnp.zeros((L,), jnp.float32)
    return 0

  lax.fori_loop(0, ZR, _zero_zbuf, 0)
  for q in range(PT // ZR):
    pltpu.sync_copy(zbuf, acc.at[pl.ds(s * PT + q * ZR, ZR)])
  plsc.subcore_barrier()

  def _chunk(i, _):
    ch = s + i * NS

    @pl.when(ch < NCH)
    def _():
      pltpu.sync_copy(h_hbm.at[pl.ds(ch * CH, CH)], hbuf)
      pltpu.sync_copy(x2_hbm.at[pl.ds(ch * IR, IR)], ibuf)
      for j in range(IR):
        for k in range(128 // L):
          v = ibuf[j, pl.ds(k * L, L)]
          in_range = (v >= lo) & (v < hi)
          ibuf[j, pl.ds(k * L, L)] = jnp.where(in_range, v - lo, HV)
      for j in range(IR):
        pltpu.sync_copy(
            hbuf.at[pl.ds(j * 128, 128)], acc.at[ibuf.at[j]], add=True
        )

    return 0

  lax.fori_loop(0, SLOTS, _chunk, 0)
  plsc.subcore_barrier()

  pltpu.sync_copy(
      acc.at[pl.ds(s * OT, OT)], out_hbm.at[pl.ds(lo + s * OT, OT)]
  )

  @pl.when(s == 0)
  def _():
    pltpu.sync_copy(
        acc.at[pl.ds(NS * OT, HV - NS * OT)],
        out_hbm.at[pl.ds(lo + NS * OT, HV - NS * OT)],
    )


_seg_sum = pl.kernel(
    _body,
    out_type=jax.ShapeDtypeStruct((V, D), jnp.float32),
    mesh=_mesh,
    scratch_types=[
        pltpu.VMEM_SHARED((ACC_R, D), jnp.float32),
        pltpu.VMEM((CH, D), jnp.float32),
        pltpu.VMEM((IR, 128), jnp.int32),
        pltpu.VMEM((ZR, D), jnp.float32),
    ],
)


@jax.jit
def kernel(H, X_node):
  x2 = X_node.astype(jnp.int32).reshape(E // 128, 128)
  return _seg_sum(H, x2)
